# Initial kernel scaffold; baseline (speedup 1.0000x reference)
import functools

import jax
import jax.numpy as jnp
from jax import lax
from jax.experimental import pallas as pl
from jax.experimental.pallas import tpu as pltpu
from jax.experimental.pallas import tpu_sc as plsc

NC = 2
NS = 16
CK = 128


def _sc_segment_sum(n_pad, rows_per_sub, ch):
  mesh = plsc.VectorSubcoreMesh(core_axis_name="c", subcore_axis_name="s")

  def body(x_hbm, src_hbm, dst_hbm, out_hbm, zbuf, srcv, dstv, rows, acc, sem):
    cid = lax.axis_index("c")
    sid = lax.axis_index("s")

    zero16 = jnp.zeros((16,), jnp.float32)

    def zbody(i, _):
      for j in range(8):
        zbuf[i, pl.ds(j * 16, 16)] = zero16
      return 0

    lax.fori_loop(0, rows_per_sub, zbody, 0)
    pltpu.sync_copy(zbuf, acc.at[pl.ds(sid * rows_per_sub, rows_per_sub)])
    plsc.subcore_barrier()

    pltpu.sync_copy(src_hbm.at[cid, sid], srcv)
    pltpu.sync_copy(dst_hbm.at[cid, sid], dstv)

    def ebody(c, _):
      pltpu.async_copy(x_hbm.at[srcv.at[c]], rows, sem).wait()
      pltpu.sync_copy(rows, acc.at[dstv.at[c]], add=True)
      return 0

    lax.fori_loop(0, ch, ebody, 0)
    plsc.subcore_barrier()

    sl = pl.ds(sid * rows_per_sub, rows_per_sub)
    pltpu.sync_copy(acc.at[sl], out_hbm.at[cid, sl])

  return pl.kernel(
      body,
      out_type=jax.ShapeDtypeStruct((NC, n_pad, 128), jnp.float32),
      mesh=mesh,
      scratch_types=[
          pltpu.VMEM((rows_per_sub, 128), jnp.float32),
          pltpu.VMEM((ch, CK), jnp.int32),
          pltpu.VMEM((ch, CK), jnp.int32),
          pltpu.VMEM((CK, 128), jnp.float32),
          pltpu.VMEM_SHARED((n_pad, 128), jnp.float32),
          pltpu.SemaphoreType.DMA,
      ],
  )


def _tc_body(p_ref, w_ref, b_ref, o_ref):
  s = p_ref[0] + p_ref[1]
  o_ref[...] = jnp.dot(s, w_ref[...], preferred_element_type=jnp.float32) + b_ref[...]


def _tc_combine_matmul(p, weight, bias, n_pad):
  br = n_pad // 8
  return pl.pallas_call(
      _tc_body,
      grid=(8,),
      in_specs=[
          pl.BlockSpec((NC, br, 128), lambda i: (0, i, 0)),
          pl.BlockSpec((128, 128), lambda i: (0, 0)),
          pl.BlockSpec((1, 128), lambda i: (0, 0)),
      ],
      out_specs=pl.BlockSpec((br, 128), lambda i: (i, 0)),
      out_shape=jax.ShapeDtypeStruct((n_pad, 128), jnp.float32),
  )(p, weight, bias.reshape(1, 128))


@jax.jit
def kernel(x, edge_index, weight, bias):
  n, d = x.shape
  e = edge_index.shape[1]
  assert d == 128 and weight.shape == (128, 128)

  ch = -(-e // (NC * NS * CK))
  e_pad = NC * NS * ch * CK
  rows_per_sub = -(-(n + 1) // NS)
  n_pad = rows_per_sub * NS

  src = edge_index[0]
  dst = edge_index[1]
  pad = e_pad - e
  src_p = jnp.concatenate([src, jnp.zeros((pad,), jnp.int32)]).reshape(NC, NS, ch, CK)
  dst_p = jnp.concatenate([dst, jnp.full((pad,), n, jnp.int32)]).reshape(NC, NS, ch, CK)

  partials = _sc_segment_sum(n_pad, rows_per_sub, ch)(x, src_p, dst_p)
  out = _tc_combine_matmul(partials, weight, bias, n_pad)
  return out[:n]


# speedup vs baseline: 6.1043x; 6.1043x over previous
import jax
import jax.numpy as jnp
from jax import lax
from jax.experimental import pallas as pl
from jax.experimental.pallas import tpu as pltpu
from jax.experimental.pallas import tpu_sc as plsc

NC = 2
NS = 16
CK = 128


def _sc_segment_sum(n_pad, rows_per_sub, ch):
  mesh = plsc.VectorSubcoreMesh(core_axis_name="c", subcore_axis_name="s")

  def body(xlo_hbm, xhi_hbm, src_hbm, dst_hbm, outlo_hbm, outhi_hbm,
           zbuf, srcv, dstv, rows, acc, sem):
    cid = lax.axis_index("c")
    sid = lax.axis_index("s")

    zero16 = jnp.zeros((16,), jnp.float32)

    def zbody(i, _):
      for j in range(4):
        zbuf[i, pl.ds(j * 16, 16)] = zero16
      return 0

    lax.fori_loop(0, rows_per_sub, zbody, 0)
    sl = pl.ds(sid * rows_per_sub, rows_per_sub)
    pltpu.sync_copy(zbuf, acc.at[sl])
    plsc.subcore_barrier()

    pltpu.sync_copy(src_hbm.at[sid], srcv)
    pltpu.sync_copy(dst_hbm.at[sid], dstv)

    def run(x_ref):
      def ebody(c, _):
        pltpu.async_copy(x_ref.at[srcv.at[c]], rows, sem).wait()
        pltpu.sync_copy(rows, acc.at[dstv.at[c]], add=True)
        return 0

      lax.fori_loop(0, ch, ebody, 0)

    pl.when(cid == 0)(lambda: run(xlo_hbm))
    pl.when(cid == 1)(lambda: run(xhi_hbm))
    plsc.subcore_barrier()

    pl.when(cid == 0)(lambda: pltpu.sync_copy(acc.at[sl], outlo_hbm.at[sl]))
    pl.when(cid == 1)(lambda: pltpu.sync_copy(acc.at[sl], outhi_hbm.at[sl]))

  return pl.kernel(
      body,
      out_type=(
          jax.ShapeDtypeStruct((n_pad, 64), jnp.float32),
          jax.ShapeDtypeStruct((n_pad, 64), jnp.float32),
      ),
      mesh=mesh,
      compiler_params=pltpu.CompilerParams(use_tc_tiling_on_sc=False),
      scratch_types=[
          pltpu.VMEM((rows_per_sub, 64), jnp.float32),
          pltpu.VMEM((ch, CK), jnp.int32),
          pltpu.VMEM((ch, CK), jnp.int32),
          pltpu.VMEM((CK, 64), jnp.float32),
          pltpu.VMEM_SHARED((n_pad, 64), jnp.float32),
          pltpu.SemaphoreType.DMA,
      ],
  )


def _tc_body(plo_ref, phi_ref, w_ref, b_ref, o_ref):
  o_ref[...] = (
      jnp.dot(plo_ref[...], w_ref[0:64, :], preferred_element_type=jnp.float32)
      + jnp.dot(phi_ref[...], w_ref[64:128, :], preferred_element_type=jnp.float32)
      + b_ref[...]
  )


def _tc_combine_matmul(plo, phi, weight, bias, n_pad):
  br = 1280
  return pl.pallas_call(
      _tc_body,
      grid=(-(-n_pad // br),),
      in_specs=[
          pl.BlockSpec((br, 64), lambda i: (i, 0)),
          pl.BlockSpec((br, 64), lambda i: (i, 0)),
          pl.BlockSpec((128, 128), lambda i: (0, 0)),
          pl.BlockSpec((1, 128), lambda i: (0, 0)),
      ],
      out_specs=pl.BlockSpec((br, 128), lambda i: (i, 0)),
      out_shape=jax.ShapeDtypeStruct((n_pad, 128), jnp.float32),
  )(plo, phi, weight, bias.reshape(1, 128))


@jax.jit
def kernel(x, edge_index, weight, bias):
  n, d = x.shape
  e = edge_index.shape[1]
  assert d == 128 and weight.shape == (128, 128)

  ch = -(-e // (NS * CK))
  e_pad = NS * ch * CK
  rows_per_sub = 8 * (-(-(n + 1) // (NS * 8)))
  n_pad = rows_per_sub * NS

  src = edge_index[0]
  dst = edge_index[1]
  pad = e_pad - e
  src_p = jnp.concatenate([src, jnp.zeros((pad,), jnp.int32)]).reshape(NS, ch, CK)
  dst_p = jnp.concatenate([dst, jnp.full((pad,), n, jnp.int32)]).reshape(NS, ch, CK)
  x_lo = x[:, :64]
  x_hi = x[:, 64:]

  plo, phi = _sc_segment_sum(n_pad, rows_per_sub, ch)(x_lo, x_hi, src_p, dst_p)
  out = _tc_combine_matmul(plo, phi, weight, bias, n_pad)
  return out[:n]
